# baseline (speedup 1.0000x reference)
import jax
import jax.numpy as jnp
from jax import lax
from jax.experimental import pallas as pl
from jax.experimental.pallas import tpu as pltpu
from jax.experimental.pallas import tpu_sc as plsc

_N = 100000
_D = 128
_C = 64

_B = 400
_NBLK = _N // _B
_NW = 32
_NGRP = _B // 16


def _segsum_body(emb_hbm, logits_hbm, zeros_d_hbm, zeros_c_hbm, ones_hbm,
                 sums_out, counts_out,
                 logits_buf, emb_buf, idx_buf, ones_buf, sums_acc, counts_acc):
    cid = lax.axis_index("c")
    sid = lax.axis_index("s")
    wid = sid * 2 + cid

    @pl.when(sid == 0)
    def _():
        pltpu.sync_copy(zeros_d_hbm, sums_acc)
        pltpu.sync_copy(zeros_c_hbm, counts_acc)

    pltpu.sync_copy(ones_hbm, ones_buf)
    plsc.subcore_barrier()

    nb = 7 + (wid < _NBLK - 7 * _NW).astype(jnp.int32)

    def block_body(t, carry):
        b = t * _NW + wid
        base = b * _B
        pltpu.sync_copy(logits_hbm.at[pl.ds(base * _C, _B * _C)], logits_buf)

        def grp_body(g, c2):
            flat = (g * 16 + lax.iota(jnp.int32, 16)) * _C
            m = jnp.full((16,), -jnp.inf, jnp.float32)
            amax = jnp.zeros((16,), jnp.int32)
            for c in range(_C):
                v = plsc.load_gather(logits_buf, [flat + c])
                upd = v > m
                m = jnp.where(upd, v, m)
                amax = jnp.where(upd, c, amax)
            idx_buf[pl.ds(g * 16, 16)] = amax
            return c2

        lax.fori_loop(0, _NGRP, grp_body, 0, unroll=False)
        pltpu.sync_copy(emb_hbm.at[pl.ds(base, _B), :], emb_buf)
        pltpu.sync_copy(emb_buf, sums_acc.at[idx_buf], add=True)
        pltpu.sync_copy(ones_buf, counts_acc.at[idx_buf], add=True)
        return carry

    lax.fori_loop(0, nb, block_body, 0, unroll=False)

    plsc.subcore_barrier()

    @pl.when(sid == 0)
    def _():
        pltpu.sync_copy(sums_acc, sums_out.at[cid])
        pltpu.sync_copy(counts_acc, counts_out.at[cid])


def _segsum_call(embedding, logits):
    mesh = plsc.VectorSubcoreMesh(
        core_axis_name="c", subcore_axis_name="s", num_cores=2, num_subcores=16
    )
    f = pl.kernel(
        _segsum_body,
        out_type=[
            jax.ShapeDtypeStruct((2, _C, _D), jnp.float32),
            jax.ShapeDtypeStruct((2, _C), jnp.float32),
        ],
        mesh=mesh,
        compiler_params=pltpu.CompilerParams(needs_layout_passes=False),
        scratch_types=[
            pltpu.VMEM((_B * _C,), jnp.float32),
            pltpu.VMEM((_B, _D), jnp.float32),
            pltpu.VMEM((_B,), jnp.int32),
            pltpu.VMEM((_B,), jnp.float32),
            pltpu.VMEM_SHARED((_C, _D), jnp.float32),
            pltpu.VMEM_SHARED((_C,), jnp.float32),
        ],
    )
    zeros_d = jnp.zeros((_C, _D), jnp.float32)
    zeros_c = jnp.zeros((_C,), jnp.float32)
    ones = jnp.ones((_B,), jnp.float32)
    return f(embedding, logits.reshape(-1), zeros_d, zeros_c, ones)


def _finalize_body(sums_ref, counts_ref, centers_ref, out_ref):
    sums = sums_ref[0] + sums_ref[1]
    counts = counts_ref[0] + counts_ref[1]
    centroids = sums / jnp.maximum(counts, 1.0)[:, None]
    delta = centers_ref[...] - centroids
    dist = jnp.sqrt(jnp.sum(delta * delta, axis=-1))
    out_ref[...] = jnp.where(counts > 0, dist, 0.0)


def _finalize_call(sums_partial, counts_partial, centers):
    return pl.pallas_call(
        _finalize_body,
        out_shape=jax.ShapeDtypeStruct((_C,), jnp.float32),
    )(sums_partial, counts_partial, centers)


def kernel(embedding, centers, logits):
    sums_partial, counts_partial = _segsum_call(embedding, logits)
    return _finalize_call(sums_partial, counts_partial, centers)


# speedup vs baseline: 1.0745x; 1.0745x over previous
import jax
import jax.numpy as jnp
from jax import lax
from jax.experimental import pallas as pl
from jax.experimental.pallas import tpu as pltpu
from jax.experimental.pallas import tpu_sc as plsc

_N = 100000
_D = 128
_C = 64

_B = 400
_NBLK = _N // _B
_NW = 32
_NGRP = _B // 16


def _segsum_body(emb_hbm, logits_hbm, zeros_d_hbm, zeros_c_hbm, ones_hbm,
                 sums_out, counts_out,
                 logits_buf, emb_buf, idx_buf, ones_buf, sums_acc, counts_acc):
    cid = lax.axis_index("c")
    sid = lax.axis_index("s")
    wid = sid * 2 + cid

    @pl.when(sid == 0)
    def _():
        pltpu.sync_copy(zeros_d_hbm, sums_acc)
        pltpu.sync_copy(zeros_c_hbm, counts_acc)

    pltpu.sync_copy(ones_hbm, ones_buf)
    plsc.subcore_barrier()

    nb = 7 + (wid < _NBLK - 7 * _NW).astype(jnp.int32)

    def block_body(t, carry):
        b = t * _NW + wid
        base = b * _B
        pltpu.sync_copy(logits_hbm.at[pl.ds(base, _B), :], logits_buf)

        def grp_body(g, c2):
            rows = g * 16 + lax.iota(jnp.int32, 16)
            ms = [jnp.full((16,), -jnp.inf, jnp.float32) for _ in range(4)]
            avs = [jnp.zeros((16,), jnp.int32) for _ in range(4)]
            for c in range(16):
                for k in range(4):
                    cc = k * 16 + c
                    v = plsc.load_gather(
                        logits_buf, [rows, jnp.full((16,), cc, jnp.int32)]
                    )
                    upd = v > ms[k]
                    ms[k] = jnp.where(upd, v, ms[k])
                    avs[k] = jnp.where(upd, cc, avs[k])
            m, amax = ms[0], avs[0]
            for k in range(1, 4):
                upd = ms[k] > m
                m = jnp.where(upd, ms[k], m)
                amax = jnp.where(upd, avs[k], amax)
            idx_buf[pl.ds(g * 16, 16)] = amax
            return c2

        lax.fori_loop(0, _NGRP, grp_body, 0, unroll=False)
        pltpu.sync_copy(emb_hbm.at[pl.ds(base, _B), :], emb_buf)
        pltpu.sync_copy(emb_buf, sums_acc.at[idx_buf], add=True)
        pltpu.sync_copy(ones_buf, counts_acc.at[idx_buf], add=True)
        return carry

    lax.fori_loop(0, nb, block_body, 0, unroll=False)

    plsc.subcore_barrier()

    @pl.when(sid == 0)
    def _():
        pltpu.sync_copy(sums_acc, sums_out.at[cid])
        pltpu.sync_copy(counts_acc, counts_out.at[cid])


def _segsum_call(embedding, logits):
    mesh = plsc.VectorSubcoreMesh(
        core_axis_name="c", subcore_axis_name="s", num_cores=2, num_subcores=16
    )
    f = pl.kernel(
        _segsum_body,
        out_type=[
            jax.ShapeDtypeStruct((2, _C, _D), jnp.float32),
            jax.ShapeDtypeStruct((2, _C), jnp.float32),
        ],
        mesh=mesh,
        compiler_params=pltpu.CompilerParams(needs_layout_passes=False),
        scratch_types=[
            pltpu.VMEM((_B, _C), jnp.float32),
            pltpu.VMEM((_B, _D), jnp.float32),
            pltpu.VMEM((_B,), jnp.int32),
            pltpu.VMEM((_B,), jnp.float32),
            pltpu.VMEM_SHARED((_C, _D), jnp.float32),
            pltpu.VMEM_SHARED((_C,), jnp.float32),
        ],
    )
    zeros_d = jnp.zeros((_C, _D), jnp.float32)
    zeros_c = jnp.zeros((_C,), jnp.float32)
    ones = jnp.ones((_B,), jnp.float32)
    return f(embedding, logits, zeros_d, zeros_c, ones)


def _finalize_body(sums_ref, counts_ref, centers_ref, out_ref):
    sums = sums_ref[0] + sums_ref[1]
    counts = counts_ref[0] + counts_ref[1]
    centroids = sums / jnp.maximum(counts, 1.0)[:, None]
    delta = centers_ref[...] - centroids
    dist = jnp.sqrt(jnp.sum(delta * delta, axis=-1))
    out_ref[...] = jnp.where(counts > 0, dist, 0.0)


def _finalize_call(sums_partial, counts_partial, centers):
    return pl.pallas_call(
        _finalize_body,
        out_shape=jax.ShapeDtypeStruct((_C,), jnp.float32),
    )(sums_partial, counts_partial, centers)


def kernel(embedding, centers, logits):
    sums_partial, counts_partial = _segsum_call(embedding, logits)
    return _finalize_call(sums_partial, counts_partial, centers)


# speedup vs baseline: 1.0814x; 1.0064x over previous
import jax
import jax.numpy as jnp
from jax import lax
from jax.experimental import pallas as pl
from jax.experimental.pallas import tpu as pltpu
from jax.experimental.pallas import tpu_sc as plsc

_N = 100000
_D = 128
_C = 64

_B = 400
_NBLK = _N // _B
_NW = 32
_NGRP = _B // 16


def _segsum_body(emb_hbm, logits_hbm, zeros_d_hbm, zeros_c_hbm, ones_hbm,
                 sums_out, counts_out,
                 logits_buf, emb_buf, idx_buf, ones_buf, sums_acc, counts_acc,
                 emb_sem):
    cid = lax.axis_index("c")
    sid = lax.axis_index("s")
    wid = sid * 2 + cid

    pltpu.sync_copy(zeros_d_hbm, sums_acc.at[sid])
    pltpu.sync_copy(zeros_c_hbm, counts_acc.at[sid])
    pltpu.sync_copy(ones_hbm, ones_buf)

    nb = 7 + (wid < _NBLK - 7 * _NW).astype(jnp.int32)

    def block_body(t, carry):
        b = t * _NW + wid
        base = b * _B
        emb_cp = pltpu.make_async_copy(
            emb_hbm.at[pl.ds(base, _B), :], emb_buf, emb_sem
        )
        emb_cp.start()
        pltpu.sync_copy(logits_hbm.at[pl.ds(base, _B), :], logits_buf)

        def grp_body(g, c2):
            rows = g * 16 + lax.iota(jnp.int32, 16)
            ms = [jnp.full((16,), -jnp.inf, jnp.float32) for _ in range(4)]
            avs = [jnp.zeros((16,), jnp.int32) for _ in range(4)]
            for c in range(16):
                for k in range(4):
                    cc = k * 16 + c
                    v = plsc.load_gather(
                        logits_buf, [rows, jnp.full((16,), cc, jnp.int32)]
                    )
                    upd = v > ms[k]
                    ms[k] = jnp.where(upd, v, ms[k])
                    avs[k] = jnp.where(upd, cc, avs[k])
            m, amax = ms[0], avs[0]
            for k in range(1, 4):
                upd = ms[k] > m
                m = jnp.where(upd, ms[k], m)
                amax = jnp.where(upd, avs[k], amax)
            idx_buf[pl.ds(g * 16, 16)] = amax
            return c2

        lax.fori_loop(0, _NGRP, grp_body, 0, unroll=False)
        emb_cp.wait()
        pltpu.sync_copy(emb_buf, sums_acc.at[sid].at[idx_buf], add=True)
        pltpu.sync_copy(ones_buf, counts_acc.at[sid].at[idx_buf], add=True)
        return carry

    lax.fori_loop(0, nb, block_body, 0, unroll=False)

    pltpu.sync_copy(sums_acc.at[sid], sums_out.at[wid])
    pltpu.sync_copy(counts_acc.at[sid], counts_out.at[wid])


def _segsum_call(embedding, logits):
    mesh = plsc.VectorSubcoreMesh(
        core_axis_name="c", subcore_axis_name="s", num_cores=2, num_subcores=16
    )
    f = pl.kernel(
        _segsum_body,
        out_type=[
            jax.ShapeDtypeStruct((_NW, _C, _D), jnp.float32),
            jax.ShapeDtypeStruct((_NW, _C), jnp.float32),
        ],
        mesh=mesh,
        compiler_params=pltpu.CompilerParams(needs_layout_passes=False),
        scratch_types=[
            pltpu.VMEM((_B, _C), jnp.float32),
            pltpu.VMEM((_B, _D), jnp.float32),
            pltpu.VMEM((_B,), jnp.int32),
            pltpu.VMEM((_B,), jnp.float32),
            pltpu.VMEM_SHARED((16, _C, _D), jnp.float32),
            pltpu.VMEM_SHARED((16, _C), jnp.float32),
            pltpu.SemaphoreType.DMA,
        ],
    )
    zeros_d = jnp.zeros((_C, _D), jnp.float32)
    zeros_c = jnp.zeros((_C,), jnp.float32)
    ones = jnp.ones((_B,), jnp.float32)
    return f(embedding, logits, zeros_d, zeros_c, ones)


def _finalize_body(sums_ref, counts_ref, centers_ref, out_ref):
    sums = jnp.sum(sums_ref[...], axis=0)
    counts = jnp.sum(counts_ref[...], axis=0)
    centroids = sums / jnp.maximum(counts, 1.0)[:, None]
    delta = centers_ref[...] - centroids
    dist = jnp.sqrt(jnp.sum(delta * delta, axis=-1))
    out_ref[...] = jnp.where(counts > 0, dist, 0.0)


def _finalize_call(sums_partial, counts_partial, centers):
    return pl.pallas_call(
        _finalize_body,
        out_shape=jax.ShapeDtypeStruct((_C,), jnp.float32),
    )(sums_partial, counts_partial, centers)


def kernel(embedding, centers, logits):
    sums_partial, counts_partial = _segsum_call(embedding, logits)
    return _finalize_call(sums_partial, counts_partial, centers)


# speedup vs baseline: 1.2388x; 1.1455x over previous
import jax
import jax.numpy as jnp
from jax import lax
from jax.experimental import pallas as pl
from jax.experimental.pallas import tpu as pltpu
from jax.experimental.pallas import tpu_sc as plsc

_N = 100000
_D = 128
_C = 64

_B = 160
_NBLK = _N // _B
_NW = 32
_NGRP = _B // 16
_NLOOPS = -(-_NBLK // _NW)


def _argmax16(logits_buf, g):
    rows = g * 16 + lax.iota(jnp.int32, 16)
    ms = [jnp.full((16,), -jnp.inf, jnp.float32) for _ in range(4)]
    avs = [jnp.zeros((16,), jnp.int32) for _ in range(4)]
    for c in range(16):
        for k in range(4):
            cc = k * 16 + c
            v = plsc.load_gather(
                logits_buf, [rows, jnp.full((16,), cc, jnp.int32)]
            )
            upd = v > ms[k]
            ms[k] = jnp.where(upd, v, ms[k])
            avs[k] = jnp.where(upd, cc, avs[k])
    m, amax = ms[0], avs[0]
    for k in range(1, 4):
        upd = ms[k] > m
        m = jnp.where(upd, ms[k], m)
        amax = jnp.where(upd, avs[k], amax)
    return amax


def _segsum_body(emb_hbm, logits_hbm, zeros_d_hbm, zeros_c_hbm, ones_hbm,
                 sums_out, counts_out,
                 logits_buf, emb_a, emb_b, ones_buf, sums_acc, counts_acc,
                 sem_a, sem_b, sem_s, sem_c):
    cid = lax.axis_index("c")
    sid = lax.axis_index("s")
    wid = sid * 2 + cid

    pltpu.sync_copy(zeros_d_hbm, sums_acc.at[sid])
    pltpu.sync_copy(zeros_c_hbm, counts_acc.at[sid])
    pltpu.sync_copy(ones_hbm, ones_buf)

    my_sums = sums_acc.at[sid]
    my_counts = counts_acc.at[sid]

    def emb_cp(t, buf, sem):
        base = jnp.minimum(t * _NW + wid, _NBLK - 1) * _B
        return pltpu.make_async_copy(emb_hbm.at[pl.ds(base, _B), :], buf, sem)

    @pl.when(wid < _NBLK)
    def _():
        emb_cp(0, emb_a, sem_a).start()

    def process(t, cur, cur_sem, nxt, nxt_sem):
        b = t * _NW + wid

        @pl.when(b < _NBLK)
        def _():
            base = b * _B
            pltpu.sync_copy(logits_hbm.at[pl.ds(base, _B), :], logits_buf)
            emb_cp(t, cur, cur_sem).wait()

            @pl.when((t + 1) * _NW + wid < _NBLK)
            def _():
                emb_cp(t + 1, nxt, nxt_sem).start()

            def grp_body(g, carry):
                amax = _argmax16(logits_buf, g)
                pltpu.make_async_copy(
                    cur.at[pl.ds(g * 16, 16), :], my_sums.at[amax], sem_s
                ).start(add=True)
                pltpu.make_async_copy(
                    ones_buf, my_counts.at[amax], sem_c
                ).start(add=True)
                return carry

            lax.fori_loop(0, _NGRP, grp_body, 0, unroll=False)

            zid = jnp.zeros((16,), jnp.int32)
            for _g in range(_NGRP):
                pltpu.make_async_copy(
                    cur.at[pl.ds(0, 16), :], my_sums.at[zid], sem_s
                ).wait()
                pltpu.make_async_copy(
                    ones_buf, my_counts.at[zid], sem_c
                ).wait()

    def pair_body(u, carry):
        process(2 * u, emb_a, sem_a, emb_b, sem_b)
        process(2 * u + 1, emb_b, sem_b, emb_a, sem_a)
        return carry

    lax.fori_loop(0, _NLOOPS // 2, pair_body, 0, unroll=False)

    pltpu.sync_copy(sums_acc.at[sid], sums_out.at[wid])
    pltpu.sync_copy(counts_acc.at[sid], counts_out.at[wid])


def _segsum_call(embedding, logits):
    mesh = plsc.VectorSubcoreMesh(
        core_axis_name="c", subcore_axis_name="s", num_cores=2, num_subcores=16
    )
    f = pl.kernel(
        _segsum_body,
        out_type=[
            jax.ShapeDtypeStruct((_NW, _C, _D), jnp.float32),
            jax.ShapeDtypeStruct((_NW, _C), jnp.float32),
        ],
        mesh=mesh,
        compiler_params=pltpu.CompilerParams(needs_layout_passes=False),
        scratch_types=[
            pltpu.VMEM((_B, _C), jnp.float32),
            pltpu.VMEM((_B, _D), jnp.float32),
            pltpu.VMEM((_B, _D), jnp.float32),
            pltpu.VMEM((16,), jnp.float32),
            pltpu.VMEM_SHARED((16, _C, _D), jnp.float32),
            pltpu.VMEM_SHARED((16, _C), jnp.float32),
            pltpu.SemaphoreType.DMA,
            pltpu.SemaphoreType.DMA,
            pltpu.SemaphoreType.DMA,
            pltpu.SemaphoreType.DMA,
        ],
    )
    zeros_d = jnp.zeros((_C, _D), jnp.float32)
    zeros_c = jnp.zeros((_C,), jnp.float32)
    ones = jnp.ones((16,), jnp.float32)
    return f(embedding, logits, zeros_d, zeros_c, ones)


def _finalize_body(sums_ref, counts_ref, centers_ref, out_ref):
    sums = jnp.sum(sums_ref[...], axis=0)
    counts = jnp.sum(counts_ref[...], axis=0)
    centroids = sums / jnp.maximum(counts, 1.0)[:, None]
    delta = centers_ref[...] - centroids
    dist = jnp.sqrt(jnp.sum(delta * delta, axis=-1))
    out_ref[...] = jnp.where(counts > 0, dist, 0.0)


def _finalize_call(sums_partial, counts_partial, centers):
    return pl.pallas_call(
        _finalize_body,
        out_shape=jax.ShapeDtypeStruct((_C,), jnp.float32),
    )(sums_partial, counts_partial, centers)


def kernel(embedding, centers, logits):
    sums_partial, counts_partial = _segsum_call(embedding, logits)
    return _finalize_call(sums_partial, counts_partial, centers)


# speedup vs baseline: 1.2675x; 1.0232x over previous
import jax
import jax.numpy as jnp
from jax import lax
from jax.experimental import pallas as pl
from jax.experimental.pallas import tpu as pltpu
from jax.experimental.pallas import tpu_sc as plsc

_N = 100000
_D = 128
_C = 64

_B = 160
_NBLK = _N // _B
_NW = 32
_NGRP = _B // 16
_NLOOPS = -(-_NBLK // _NW)


def _argmax16(logits_buf, g):
    rows = g * 16 + lax.iota(jnp.int32, 16)
    ms = [jnp.full((16,), -jnp.inf, jnp.float32) for _ in range(4)]
    avs = [jnp.zeros((16,), jnp.int32) for _ in range(4)]
    for c in range(16):
        for k in range(4):
            cc = k * 16 + c
            v = plsc.load_gather(
                logits_buf, [rows, jnp.full((16,), cc, jnp.int32)]
            )
            upd = v > ms[k]
            ms[k] = jnp.where(upd, v, ms[k])
            avs[k] = jnp.where(upd, cc, avs[k])
    m, amax = ms[0], avs[0]
    for k in range(1, 4):
        upd = ms[k] > m
        m = jnp.where(upd, ms[k], m)
        amax = jnp.where(upd, avs[k], amax)
    return amax


def _segsum_body(emb_hbm, logits_hbm, zeros_d_hbm,
                 sums_out, counts_out,
                 logits_buf, emb_a, emb_b, counts_buf, sums_acc,
                 sem_a, sem_b, sem_s):
    cid = lax.axis_index("c")
    sid = lax.axis_index("s")
    wid = sid * 2 + cid

    pltpu.sync_copy(zeros_d_hbm, sums_acc.at[sid])
    for j in range(_C // 16):
        counts_buf[pl.ds(j * 16, 16)] = jnp.zeros((16,), jnp.float32)

    my_sums = sums_acc.at[sid]

    def emb_cp(t, buf, sem):
        base = jnp.minimum(t * _NW + wid, _NBLK - 1) * _B
        return pltpu.make_async_copy(emb_hbm.at[pl.ds(base, _B), :], buf, sem)

    @pl.when(wid < _NBLK)
    def _():
        emb_cp(0, emb_a, sem_a).start()

    def process(t, cur, cur_sem, nxt, nxt_sem):
        b = t * _NW + wid

        @pl.when(b < _NBLK)
        def _():
            base = b * _B
            pltpu.sync_copy(logits_hbm.at[pl.ds(base, _B), :], logits_buf)
            emb_cp(t, cur, cur_sem).wait()

            @pl.when((t + 1) * _NW + wid < _NBLK)
            def _():
                emb_cp(t + 1, nxt, nxt_sem).start()

            def grp_body(g, carry):
                amax = _argmax16(logits_buf, g)
                pltpu.make_async_copy(
                    cur.at[pl.ds(g * 16, 16), :], my_sums.at[amax], sem_s
                ).start(add=True)
                plsc.addupdate_scatter(
                    counts_buf, [amax], jnp.full((16,), 1.0, jnp.float32)
                )
                return carry

            lax.fori_loop(0, _NGRP, grp_body, 0, unroll=False)

            zid = jnp.zeros((16,), jnp.int32)
            for _g in range(_NGRP):
                pltpu.make_async_copy(
                    cur.at[pl.ds(0, 16), :], my_sums.at[zid], sem_s
                ).wait()

    def pair_body(u, carry):
        process(2 * u, emb_a, sem_a, emb_b, sem_b)
        process(2 * u + 1, emb_b, sem_b, emb_a, sem_a)
        return carry

    lax.fori_loop(0, _NLOOPS // 2, pair_body, 0, unroll=False)

    pltpu.sync_copy(sums_acc.at[sid], sums_out.at[wid])
    pltpu.sync_copy(counts_buf, counts_out.at[wid])


def _segsum_call(embedding, logits):
    mesh = plsc.VectorSubcoreMesh(
        core_axis_name="c", subcore_axis_name="s", num_cores=2, num_subcores=16
    )
    f = pl.kernel(
        _segsum_body,
        out_type=[
            jax.ShapeDtypeStruct((_NW, _C, _D), jnp.float32),
            jax.ShapeDtypeStruct((_NW, _C), jnp.float32),
        ],
        mesh=mesh,
        compiler_params=pltpu.CompilerParams(needs_layout_passes=False),
        scratch_types=[
            pltpu.VMEM((_B, _C), jnp.float32),
            pltpu.VMEM((_B, _D), jnp.float32),
            pltpu.VMEM((_B, _D), jnp.float32),
            pltpu.VMEM((_C,), jnp.float32),
            pltpu.VMEM_SHARED((16, _C, _D), jnp.float32),
            pltpu.SemaphoreType.DMA,
            pltpu.SemaphoreType.DMA,
            pltpu.SemaphoreType.DMA,
        ],
    )
    zeros_d = jnp.zeros((_C, _D), jnp.float32)
    return f(embedding, logits, zeros_d)


def _finalize_body(sums_ref, counts_ref, centers_ref, out_ref):
    sums = jnp.sum(sums_ref[...], axis=0)
    counts = jnp.sum(counts_ref[...], axis=0)
    centroids = sums / jnp.maximum(counts, 1.0)[:, None]
    delta = centers_ref[...] - centroids
    dist = jnp.sqrt(jnp.sum(delta * delta, axis=-1))
    out_ref[...] = jnp.where(counts > 0, dist, 0.0)


def _finalize_call(sums_partial, counts_partial, centers):
    return pl.pallas_call(
        _finalize_body,
        out_shape=jax.ShapeDtypeStruct((_C,), jnp.float32),
    )(sums_partial, counts_partial, centers)


def kernel(embedding, centers, logits):
    sums_partial, counts_partial = _segsum_call(embedding, logits)
    return _finalize_call(sums_partial, counts_partial, centers)
